# Initial kernel scaffold; baseline (speedup 1.0000x reference)
import jax
import jax.numpy as jnp
from jax.experimental import pallas as pl


def kernel(imgs, heatmaps, extra, wT, b):
    raise NotImplementedError("write your pallas kernel here")


# speedup vs baseline: 6.7019x; 6.7019x over previous
import functools

import jax
import jax.numpy as jnp
from jax.experimental import pallas as pl
from jax.experimental.pallas import tpu as pltpu


def _fused_mse_kernel(x_ref, w_ref, b_ref, gt_ref, pred_ref, lpart_ref, *,
                      hw_valid, padded):
    x = x_ref[0]
    w = w_ref[...]
    pred = jax.lax.dot_general(
        w, x, (((1,), (0,)), ((), ())),
        preferred_element_type=jnp.float32)
    pred = pred + b_ref[...]
    pred_ref[0] = pred
    d = pred - gt_ref[0]
    sq = d * d
    if padded:
        pos = jax.lax.broadcasted_iota(jnp.int32, sq.shape, 1)
        sq = jnp.where(pos < hw_valid, sq, 0.0)
    lpart_ref[...] = jnp.broadcast_to(jnp.sum(sq), lpart_ref.shape)


def _fused_call(x, wT, b, gt, hw):
    n, cin, hwp = x.shape
    k = wT.shape[0]
    kern = functools.partial(_fused_mse_kernel, hw_valid=hw,
                             padded=(hwp != hw))
    return pl.pallas_call(
        kern,
        out_shape=(jax.ShapeDtypeStruct((n, k, hwp), jnp.float32),
                   jax.ShapeDtypeStruct((n, 1, 128), jnp.float32)),
        grid=(n,),
        in_specs=[
            pl.BlockSpec((1, cin, hwp), lambda i: (i, 0, 0)),
            pl.BlockSpec((k, cin), lambda i: (0, 0)),
            pl.BlockSpec((k, 1), lambda i: (0, 0)),
            pl.BlockSpec((1, k, hwp), lambda i: (i, 0, 0)),
        ],
        out_specs=(
            pl.BlockSpec((1, k, hwp), lambda i: (i, 0, 0)),
            pl.BlockSpec((1, 1, 128), lambda i: (i, 0, 0)),
        ),
        compiler_params=pltpu.CompilerParams(
            dimension_semantics=("parallel",)),
    )(x, wT, b, gt)


def kernel(imgs, heatmaps, extra, wT, b):
    n, c, h, w = imgs.shape
    k = wT.shape[0]
    hw = h * w
    hwp = -(-hw // 128) * 128
    x = imgs.reshape(n, c, hw)
    gt = heatmaps.reshape(n, k, hw)
    if hwp != hw:
        x = jnp.pad(x, ((0, 0), (0, 0), (0, hwp - hw)))
        gt = jnp.pad(gt, ((0, 0), (0, 0), (0, hwp - hw)))
    pred, lpart = _fused_call(x, wT, b, gt, hw)
    if hwp != hw:
        pred = pred[:, :, :hw]
    loss = jnp.sum(lpart[:, 0, 0]) * (1.0 / float(heatmaps.size))
    return [pred.reshape(n, k, h, w), loss]
